# baseline (speedup 1.0000x reference)
import functools

import jax
import jax.numpy as jnp
from jax import lax
from jax.experimental import pallas as pl
from jax.experimental.pallas import tpu as pltpu
from jax.experimental.pallas import tpu_sc as plsc

H = 2048
E = 8
K = 2
I = 2048
T = 2048

BLK = 256
PMAX = T * K + E * BLK
NB = PMAX // BLK
IB = 1024
HB = 1024
NI = I // IB
NH = H // HB

NC = 2
NS = 16
NW = NC * NS
TW = T // NW
CH = 16
NCH = TW // CH


def _routing_body(x_ref, wg_ref, slots_ref, wts_ref, bexp_ref):
    x = x_ref[...]
    logits = jnp.dot(x, wg_ref[...], preferred_element_type=jnp.float32)
    iota_e = lax.broadcasted_iota(jnp.int32, (T, E), 1)
    m1 = jnp.max(logits, axis=1, keepdims=True)
    i1 = jnp.min(jnp.where(logits == m1, iota_e, E), axis=1, keepdims=True)
    masked = jnp.where(iota_e == i1, jnp.float32(-jnp.inf), logits)
    m2 = jnp.max(masked, axis=1, keepdims=True)
    i2 = jnp.min(jnp.where(masked == m2, iota_e, E), axis=1, keepdims=True)
    w0 = lax.logistic(m1 - m2)
    w1v = 1.0 - w0

    oh1 = (iota_e == i1).astype(jnp.float32)
    oh2 = (iota_e == i2).astype(jnp.float32)
    both = oh1 + oh2
    ri = lax.broadcasted_iota(jnp.int32, (T, 1), 0)
    ci = lax.broadcasted_iota(jnp.int32, (1, T), 1)
    mstrict = (ci < ri).astype(jnp.float32)
    excl = jnp.dot(mstrict, both, preferred_element_type=jnp.float32)
    counts = jnp.sum(both, axis=0, keepdims=True)
    padded = jnp.ceil(counts / BLK) * BLK
    er = lax.broadcasted_iota(jnp.int32, (E, 1), 0)
    ec = lax.broadcasted_iota(jnp.int32, (1, E), 1)
    tri = (er < ec).astype(jnp.float32)
    offs = jnp.dot(padded, tri, preferred_element_type=jnp.float32)
    slot_base = offs + excl
    slot0 = jnp.sum(oh1 * slot_base, axis=1).astype(jnp.int32)
    slot1 = jnp.sum(oh2 * slot_base, axis=1).astype(jnp.int32)
    ends = offs + padded
    bstart = (lax.broadcasted_iota(jnp.int32, (NB, E), 0) * BLK).astype(jnp.float32)
    bexp = jnp.sum((bstart >= ends).astype(jnp.int32), axis=1)

    slots_ref[0, :] = slot0
    slots_ref[1, :] = slot1
    wts_ref[0, :] = w0[:, 0]
    wts_ref[1, :] = w1v[:, 0]
    bexp_ref[0, :] = bexp


_routing_call = pl.pallas_call(
    _routing_body,
    out_shape=(
        jax.ShapeDtypeStruct((K, T), jnp.int32),
        jax.ShapeDtypeStruct((K, T), jnp.float32),
        jax.ShapeDtypeStruct((1, NB), jnp.int32),
    ),
)


def _k1_body(s_ref, x_ref, w1_ref, w3_ref, o_ref):
    e = s_ref[pl.program_id(1)]

    @pl.when(e < E)
    def _():
        xb = x_ref[...].astype(jnp.bfloat16)
        g = lax.dot_general(xb, w1_ref[0].astype(jnp.bfloat16),
                            (((1,), (1,)), ((), ())),
                            preferred_element_type=jnp.float32)
        u = lax.dot_general(xb, w3_ref[0].astype(jnp.bfloat16),
                            (((1,), (1,)), ((), ())),
                            preferred_element_type=jnp.float32)
        o_ref[...] = g * lax.logistic(g) * u


_k1_call = pl.pallas_call(
    _k1_body,
    grid_spec=pltpu.PrefetchScalarGridSpec(
        num_scalar_prefetch=1,
        grid=(NI, NB),
        in_specs=[
            pl.BlockSpec((BLK, H), lambda i, b, s: (b, 0)),
            pl.BlockSpec((1, IB, H), lambda i, b, s: (jnp.minimum(s[b], E - 1), i, 0)),
            pl.BlockSpec((1, IB, H), lambda i, b, s: (jnp.minimum(s[b], E - 1), i, 0)),
        ],
        out_specs=pl.BlockSpec((BLK, IB), lambda i, b, s: (b, i)),
    ),
    out_shape=jax.ShapeDtypeStruct((PMAX, I), jnp.float32),
)


def _k2_body(s_ref, a_ref, w2_ref, o_ref):
    e = s_ref[pl.program_id(1)]

    @pl.when(e < E)
    def _():
        o_ref[...] = lax.dot_general(a_ref[...].astype(jnp.bfloat16),
                                     w2_ref[0].astype(jnp.bfloat16),
                                     (((1,), (1,)), ((), ())),
                                     preferred_element_type=jnp.float32)


_k2_call = pl.pallas_call(
    _k2_body,
    grid_spec=pltpu.PrefetchScalarGridSpec(
        num_scalar_prefetch=1,
        grid=(NH, NB),
        in_specs=[
            pl.BlockSpec((BLK, I), lambda h, b, s: (b, 0)),
            pl.BlockSpec((1, HB, I), lambda h, b, s: (jnp.minimum(s[b], E - 1), h, 0)),
        ],
        out_specs=pl.BlockSpec((BLK, HB), lambda h, b, s: (b, h)),
    ),
    out_shape=jax.ShapeDtypeStruct((PMAX, H), jnp.float32),
)


def _dispatch_call(x, slots):
    mesh = plsc.VectorSubcoreMesh(core_axis_name="c", subcore_axis_name="s")

    @functools.partial(
        pl.kernel,
        out_type=jax.ShapeDtypeStruct((PMAX, H), jnp.float32),
        mesh=mesh,
        scratch_types=[
            pltpu.VMEM((CH, H), jnp.float32),
            pltpu.VMEM((CH,), jnp.int32),
            pltpu.VMEM((CH,), jnp.int32),
            pltpu.SemaphoreType.DMA,
            pltpu.SemaphoreType.DMA,
        ],
    )
    def dk(x_hbm, slots_hbm, xs_hbm, rows_v, idx0_v, idx1_v, sem0, sem1):
        wid = lax.axis_index("s") * NC + lax.axis_index("c")
        for c in range(NCH):
            base = wid * TW + c * CH
            pltpu.sync_copy(x_hbm.at[pl.ds(base, CH)], rows_v)
            pltpu.sync_copy(slots_hbm.at[0, pl.ds(base, CH)], idx0_v)
            pltpu.sync_copy(slots_hbm.at[1, pl.ds(base, CH)], idx1_v)
            cp0 = pltpu.async_copy(rows_v, xs_hbm.at[idx0_v], sem0)
            cp1 = pltpu.async_copy(rows_v, xs_hbm.at[idx1_v], sem1)
            cp0.wait()
            cp1.wait()

    return dk(x, slots)


def _combine_call(y, slots, wts):
    mesh = plsc.VectorSubcoreMesh(core_axis_name="c", subcore_axis_name="s")

    @functools.partial(
        pl.kernel,
        out_type=jax.ShapeDtypeStruct((T, H), jnp.float32),
        mesh=mesh,
        scratch_types=[
            pltpu.VMEM((CH, H), jnp.float32),
            pltpu.VMEM((CH, H), jnp.float32),
            pltpu.VMEM((CH, H), jnp.float32),
            pltpu.VMEM((CH,), jnp.int32),
            pltpu.VMEM((CH,), jnp.int32),
            pltpu.VMEM((CH,), jnp.float32),
            pltpu.VMEM((CH,), jnp.float32),
            pltpu.SemaphoreType.DMA,
            pltpu.SemaphoreType.DMA,
        ],
    )
    def ck(y_hbm, slots_hbm, wts_hbm, out_hbm,
           r0_v, r1_v, o_v, i0_v, i1_v, w0_v, w1_v, sem0, sem1):
        wid = lax.axis_index("s") * NC + lax.axis_index("c")
        for c in range(NCH):
            base = wid * TW + c * CH
            pltpu.sync_copy(slots_hbm.at[0, pl.ds(base, CH)], i0_v)
            pltpu.sync_copy(slots_hbm.at[1, pl.ds(base, CH)], i1_v)
            pltpu.sync_copy(wts_hbm.at[0, pl.ds(base, CH)], w0_v)
            pltpu.sync_copy(wts_hbm.at[1, pl.ds(base, CH)], w1_v)
            cp0 = pltpu.async_copy(y_hbm.at[i0_v], r0_v, sem0)
            cp1 = pltpu.async_copy(y_hbm.at[i1_v], r1_v, sem1)
            cp0.wait()
            cp1.wait()
            wvec0 = w0_v[...]
            wvec1 = w1_v[...]
            for r in range(CH):
                s0 = wvec0[r]
                s1 = wvec1[r]

                def body(j, _):
                    cs = pl.ds(j * 16, 16)
                    o_v[r, cs] = r0_v[r, cs] * s0 + r1_v[r, cs] * s1
                    return 0

                lax.fori_loop(0, H // 16, body, 0)
            pltpu.sync_copy(o_v, out_hbm.at[pl.ds(base, CH)])

    return ck(y, slots, wts)


def kernel(hidden_states, w_gate, w1, w3, w2):
    x = hidden_states.reshape(T, H)
    slots, wts, bexp2d = _routing_call(x, w_gate)
    bexp = bexp2d.reshape(NB)
    xs = _dispatch_call(x, slots)
    a = _k1_call(bexp, xs, w1, w3)
    y = _k2_call(bexp, a, w2)
    out = _combine_call(y, slots, wts)
    return out.reshape(hidden_states.shape)


# speedup vs baseline: 1.0705x; 1.0705x over previous
import functools

import jax
import jax.numpy as jnp
from jax import lax
from jax.experimental import pallas as pl
from jax.experimental.pallas import tpu as pltpu
from jax.experimental.pallas import tpu_sc as plsc

H = 2048
E = 8
K = 2
I = 2048
T = 2048

BLK = 256
PMAX = T * K + E * BLK
NB = PMAX // BLK
IB = 1024
HB = 2048
NI = I // IB
NH = H // HB

NC = 2
NS = 16
NW = NC * NS
TW = T // NW
CH = 16
NCH = TW // CH


def _routing_body(x_ref, wg_ref, slots_ref, wts_ref, bexp_ref):
    x = x_ref[...]
    logits = jnp.dot(x, wg_ref[...], preferred_element_type=jnp.float32)
    iota_e = lax.broadcasted_iota(jnp.int32, (T, E), 1)
    m1 = jnp.max(logits, axis=1, keepdims=True)
    i1 = jnp.min(jnp.where(logits == m1, iota_e, E), axis=1, keepdims=True)
    masked = jnp.where(iota_e == i1, jnp.float32(-jnp.inf), logits)
    m2 = jnp.max(masked, axis=1, keepdims=True)
    i2 = jnp.min(jnp.where(masked == m2, iota_e, E), axis=1, keepdims=True)
    w0 = lax.logistic(m1 - m2)
    w1v = 1.0 - w0

    oh1 = (iota_e == i1).astype(jnp.float32)
    oh2 = (iota_e == i2).astype(jnp.float32)
    both = oh1 + oh2
    ri = lax.broadcasted_iota(jnp.int32, (T, 1), 0)
    ci = lax.broadcasted_iota(jnp.int32, (1, T), 1)
    mstrict = (ci < ri).astype(jnp.float32)
    excl = jnp.dot(mstrict, both, preferred_element_type=jnp.float32)
    counts = jnp.sum(both, axis=0, keepdims=True)
    padded = jnp.ceil(counts / BLK) * BLK
    er = lax.broadcasted_iota(jnp.int32, (E, 1), 0)
    ec = lax.broadcasted_iota(jnp.int32, (1, E), 1)
    tri = (er < ec).astype(jnp.float32)
    offs = jnp.dot(padded, tri, preferred_element_type=jnp.float32)
    slot_base = offs + excl
    slot0 = jnp.sum(oh1 * slot_base, axis=1).astype(jnp.int32)
    slot1 = jnp.sum(oh2 * slot_base, axis=1).astype(jnp.int32)
    ends = offs + padded
    bstart = (lax.broadcasted_iota(jnp.int32, (NB, E), 0) * BLK).astype(jnp.float32)
    bexp = jnp.sum((bstart >= ends).astype(jnp.int32), axis=1)

    slots_ref[0, :] = slot0
    slots_ref[1, :] = slot1
    wts_ref[0, :] = w0[:, 0]
    wts_ref[1, :] = w1v[:, 0]
    bexp_ref[0, :] = bexp


_routing_call = pl.pallas_call(
    _routing_body,
    out_shape=(
        jax.ShapeDtypeStruct((K, T), jnp.int32),
        jax.ShapeDtypeStruct((K, T), jnp.float32),
        jax.ShapeDtypeStruct((1, NB), jnp.int32),
    ),
)


def _k1_body(s_ref, x_ref, w1_ref, w3_ref, o_ref):
    e = s_ref[pl.program_id(1)]

    @pl.when(e < E)
    def _():
        xb = x_ref[...].astype(jnp.bfloat16)
        g = lax.dot_general(xb, w1_ref[0].astype(jnp.bfloat16),
                            (((1,), (1,)), ((), ())),
                            preferred_element_type=jnp.float32)
        u = lax.dot_general(xb, w3_ref[0].astype(jnp.bfloat16),
                            (((1,), (1,)), ((), ())),
                            preferred_element_type=jnp.float32)
        o_ref[...] = (g * lax.logistic(g) * u).astype(jnp.bfloat16)


_k1_call = pl.pallas_call(
    _k1_body,
    grid_spec=pltpu.PrefetchScalarGridSpec(
        num_scalar_prefetch=1,
        grid=(NI, NB),
        in_specs=[
            pl.BlockSpec((BLK, H), lambda i, b, s: (b, 0)),
            pl.BlockSpec((1, IB, H), lambda i, b, s: (jnp.minimum(s[b], E - 1), i, 0)),
            pl.BlockSpec((1, IB, H), lambda i, b, s: (jnp.minimum(s[b], E - 1), i, 0)),
        ],
        out_specs=pl.BlockSpec((BLK, IB), lambda i, b, s: (b, i)),
    ),
    out_shape=jax.ShapeDtypeStruct((PMAX, I), jnp.bfloat16),
)


def _k2_body(s_ref, a_ref, w2_ref, o_ref):
    e = s_ref[pl.program_id(1)]

    @pl.when(e < E)
    def _():
        o_ref[...] = lax.dot_general(a_ref[...],
                                     w2_ref[0].astype(jnp.bfloat16),
                                     (((1,), (1,)), ((), ())),
                                     preferred_element_type=jnp.float32)


_k2_call = pl.pallas_call(
    _k2_body,
    grid_spec=pltpu.PrefetchScalarGridSpec(
        num_scalar_prefetch=1,
        grid=(NH, NB),
        in_specs=[
            pl.BlockSpec((BLK, I), lambda h, b, s: (b, 0)),
            pl.BlockSpec((1, HB, I), lambda h, b, s: (jnp.minimum(s[b], E - 1), h, 0)),
        ],
        out_specs=pl.BlockSpec((BLK, HB), lambda h, b, s: (b, h)),
    ),
    out_shape=jax.ShapeDtypeStruct((PMAX, H), jnp.float32),
)


def _dispatch_call(x, slots):
    mesh = plsc.VectorSubcoreMesh(core_axis_name="c", subcore_axis_name="s")

    @functools.partial(
        pl.kernel,
        out_type=jax.ShapeDtypeStruct((PMAX, H), jnp.float32),
        mesh=mesh,
        scratch_types=[
            pltpu.VMEM((CH, H), jnp.float32),
            pltpu.VMEM((CH,), jnp.int32),
            pltpu.VMEM((CH,), jnp.int32),
            pltpu.SemaphoreType.DMA,
            pltpu.SemaphoreType.DMA,
        ],
    )
    def dk(x_hbm, slots_hbm, xs_hbm, rows_v, idx0_v, idx1_v, sem0, sem1):
        wid = lax.axis_index("s") * NC + lax.axis_index("c")
        for c in range(NCH):
            base = wid * TW + c * CH
            pltpu.sync_copy(x_hbm.at[pl.ds(base, CH)], rows_v)
            pltpu.sync_copy(slots_hbm.at[0, pl.ds(base, CH)], idx0_v)
            pltpu.sync_copy(slots_hbm.at[1, pl.ds(base, CH)], idx1_v)
            cp0 = pltpu.async_copy(rows_v, xs_hbm.at[idx0_v], sem0)
            cp1 = pltpu.async_copy(rows_v, xs_hbm.at[idx1_v], sem1)
            cp0.wait()
            cp1.wait()

    return dk(x, slots)


def _combine_call(y, slots, wts):
    mesh = plsc.VectorSubcoreMesh(core_axis_name="c", subcore_axis_name="s")

    @functools.partial(
        pl.kernel,
        out_type=jax.ShapeDtypeStruct((T, H), jnp.float32),
        mesh=mesh,
        scratch_types=[
            pltpu.VMEM((CH, H), jnp.float32),
            pltpu.VMEM((CH, H), jnp.float32),
            pltpu.VMEM((CH, H), jnp.float32),
            pltpu.VMEM((CH,), jnp.int32),
            pltpu.VMEM((CH,), jnp.int32),
            pltpu.VMEM((CH,), jnp.float32),
            pltpu.VMEM((CH,), jnp.float32),
            pltpu.SemaphoreType.DMA,
            pltpu.SemaphoreType.DMA,
        ],
    )
    def ck(y_hbm, slots_hbm, wts_hbm, out_hbm,
           r0_v, r1_v, o_v, i0_v, i1_v, w0_v, w1_v, sem0, sem1):
        wid = lax.axis_index("s") * NC + lax.axis_index("c")
        for c in range(NCH):
            base = wid * TW + c * CH
            pltpu.sync_copy(slots_hbm.at[0, pl.ds(base, CH)], i0_v)
            pltpu.sync_copy(slots_hbm.at[1, pl.ds(base, CH)], i1_v)
            pltpu.sync_copy(wts_hbm.at[0, pl.ds(base, CH)], w0_v)
            pltpu.sync_copy(wts_hbm.at[1, pl.ds(base, CH)], w1_v)
            cp0 = pltpu.async_copy(y_hbm.at[i0_v], r0_v, sem0)
            cp1 = pltpu.async_copy(y_hbm.at[i1_v], r1_v, sem1)
            cp0.wait()
            cp1.wait()
            wvec0 = w0_v[...]
            wvec1 = w1_v[...]
            for r in range(CH):
                s0 = wvec0[r]
                s1 = wvec1[r]

                def body(j, _):
                    cs = pl.ds(j * 16, 16)
                    o_v[r, cs] = r0_v[r, cs] * s0 + r1_v[r, cs] * s1
                    return 0

                lax.fori_loop(0, H // 16, body, 0)
            pltpu.sync_copy(o_v, out_hbm.at[pl.ds(base, CH)])

    return ck(y, slots, wts)


def kernel(hidden_states, w_gate, w1, w3, w2):
    x = hidden_states.reshape(T, H)
    slots, wts, bexp2d = _routing_call(x, w_gate)
    bexp = bexp2d.reshape(NB)
    xs = _dispatch_call(x, slots)
    a = _k1_call(bexp, xs, w1, w3)
    y = _k2_call(bexp, a, w2)
    out = _combine_call(y, slots, wts)
    return out.reshape(hidden_states.shape)
